# baseline (speedup 1.0000x reference)
import jax
import jax.numpy as jnp
from jax.experimental import pallas as pl

_N = 100000
_BR = 1000
_GRID = _N // _BR


def _body(x_ref, c_ref, o_ref):
    o_ref[...] = x_ref[...]
    @pl.when(pl.program_id(0) == 0)
    def _():
        o_ref[0:8, :, :] = o_ref[0:8, :, :] + c_ref[...]


def kernel(x):
    t = jnp.arange(0, 336, 1, dtype=jnp.float32).reshape(7, 6, 8)
    addvals = jnp.concatenate(
        [t[jnp.array([0, 2, 1, 3, 4, 5, 6])], jnp.zeros((1, 6, 8), jnp.float32)], axis=0
    )
    return pl.pallas_call(
        _body,
        grid=(_GRID,),
        in_specs=[
            pl.BlockSpec((_BR, 6, 8), lambda i: (i, 0, 0)),
            pl.BlockSpec((8, 6, 8), lambda i: (0, 0, 0)),
        ],
        out_specs=pl.BlockSpec((_BR, 6, 8), lambda i: (i, 0, 0)),
        out_shape=jax.ShapeDtypeStruct((_N, 6, 8), jnp.float32),
    )(x, addvals)


# speedup vs baseline: 1.0000x; 1.0000x over previous
import jax
import jax.numpy as jnp
from jax.experimental import pallas as pl
from jax.experimental.pallas import tpu as pltpu

_N = 100000
_BR = 1000
_GRID = _N // _BR


def _body(x_ref, c_ref, o_ref):
    o_ref[...] = x_ref[...]
    @pl.when(pl.program_id(0) == 0)
    def _():
        o_ref[0:8, :, :] = o_ref[0:8, :, :] + c_ref[...]


def kernel(x):
    t = jnp.arange(0, 336, 1, dtype=jnp.float32).reshape(7, 6, 8)
    addvals = jnp.concatenate(
        [t[jnp.array([0, 2, 1, 3, 4, 5, 6])], jnp.zeros((1, 6, 8), jnp.float32)], axis=0
    )
    return pl.pallas_call(
        _body,
        grid=(_GRID,),
        in_specs=[
            pl.BlockSpec((_BR, 6, 8), lambda i: (i, 0, 0)),
            pl.BlockSpec((8, 6, 8), lambda i: (0, 0, 0)),
        ],
        out_specs=pl.BlockSpec((_BR, 6, 8), lambda i: (i, 0, 0)),
        out_shape=jax.ShapeDtypeStruct((_N, 6, 8), jnp.float32),
        compiler_params=pltpu.CompilerParams(
            dimension_semantics=("arbitrary",),
        ),
    )(x, addvals)
